# baseline (speedup 1.0000x reference)
import functools

import jax
import jax.numpy as jnp
from jax import lax
from jax.experimental import pallas as pl
from jax.experimental.pallas import tpu as pltpu
from jax.experimental.pallas import tpu_sc as plsc

N = 10000
D = 128
KNB = 32
R = 500
BS = 4096
ST = 4
A = float((1.0 + 1.0 / (KNB * ST)) ** ST)

L = 16
NC = 2
NS = 16
NW = NC * NS
QPW = BS // NW
DC = D // L
RP = 512


def _softmax2(v0, v1):
    m = jnp.maximum(jnp.max(v0), jnp.max(v1))
    x0 = jnp.exp(v0 - m)
    x1 = jnp.exp(v1 - m)
    s = jnp.sum(x0) + jnp.sum(x1)
    return x0 / s, x1 / s


def _sc_body(emb_hbm, nw_hbm, relw_hbm, nbr_hbm,
             e1s_hbm, r1s_hbm, e2s_hbm, e3s_hbm,
             d1_hbm, d2_hbm,
             e1i, e2i, e3i, nwb, relwb, r1b, histb, ewb,
             E1, E2, E3, NBE1, NBE2, nb1a, nb1b, nb2a, nb2b, wb, wb2, d1b, d2b,
             sem0, s1a, s1b, s2a, s2b):
    wid = lax.axis_index("s") * NC + lax.axis_index("c")
    base = wid * QPW

    pltpu.sync_copy(e1s_hbm.at[pl.ds(base, QPW)], e1i)
    pltpu.sync_copy(e2s_hbm.at[pl.ds(base, QPW)], e2i)
    pltpu.sync_copy(e3s_hbm.at[pl.ds(base, QPW)], e3i)
    pltpu.sync_copy(nw_hbm, nwb)
    pltpu.sync_copy(relw_hbm, relwb.at[pl.ds(0, R + 1)])
    pltpu.sync_copy(r1s_hbm, r1b)

    hs = [
        pltpu.async_copy(emb_hbm.at[e1i], E1, sem0),
        pltpu.async_copy(emb_hbm.at[e2i], E2, sem0),
        pltpu.async_copy(emb_hbm.at[e3i], E3, sem0),
        pltpu.async_copy(nbr_hbm.at[e1i], NBE1, sem0),
        pltpu.async_copy(nbr_hbm.at[e2i], NBE2, sem0),
    ]

    zero = jnp.zeros((L,), jnp.float32)
    one = jnp.ones((L,), jnp.float32)
    for j in range(RP // L):
        histb[pl.ds(j * L, L)] = zero

    def _hist_step(j, _):
        idx = r1b[pl.ds(j * L, L)]
        plsc.addupdate_scatter(histb, [idx], one)
        return 0

    lax.fori_loop(0, BS // L, _hist_step, 0)

    for j in range(RP // L):
        sl = pl.ds(j * L, L)
        ewb[sl] = relwb[sl] * (1.0 + histb[sl])

    for h in hs:
        h.wait()

    lane = lax.iota(jnp.int32, L)
    mask0 = lane == 0
    nb1s, nb2s = (nb1a, nb1b), (nb2a, nb2b)
    s1s, s2s = (s1a, s1b), (s2a, s2b)
    UN = 8

    def _fire(q, slot):
        pltpu.make_async_copy(emb_hbm.at[NBE1.at[q].at[pl.ds(0, KNB)]],
                              nb1s[slot], s1s[slot]).start()
        pltpu.make_async_copy(emb_hbm.at[NBE2.at[q].at[pl.ds(0, KNB)]],
                              nb2s[slot], s2s[slot]).start()

    def _accumulate(wref, nbbuf, acc0):
        def _step(j, acc):
            for u in range(UN):
                k = j * UN + u
                wk = plsc.load_gather(wref, [jnp.full((L,), k, jnp.int32)])
                acc = tuple(acc[c] + wk * nbbuf[k, pl.ds(c * L, L)]
                            for c in range(DC))
            return acc
        return lax.fori_loop(0, KNB // UN, _step, acc0)

    zacc = tuple(jnp.zeros((L,), jnp.float32) for _ in range(DC))

    _fire(jnp.int32(0), 0)

    def _pair(g, _):
        for b in (0, 1):
            i = 2 * g + b

            if b == 0:
                _fire(i + 1, 1)
            else:
                @pl.when(g < QPW // 2 - 1)
                def _():
                    _fire(i + 1, 0)

            i10 = NBE1[i, pl.ds(0, L)]
            i11 = NBE1[i, pl.ds(L, L)]
            w0, w1 = _softmax2(plsc.load_gather(nwb, [i10]),
                               plsc.load_gather(nwb, [i11]))
            wb[pl.ds(0, L)] = w0
            wb[pl.ds(L, L)] = w1
            i20 = NBE2[i, pl.ds(0, L)]
            i21 = NBE2[i, pl.ds(L, L)]
            r20 = NBE2[i, pl.ds(KNB, L)]
            r21 = NBE2[i, pl.ds(KNB + L, L)]
            v0, v1 = _softmax2(
                plsc.load_gather(nwb, [i20]) + plsc.load_gather(ewb, [r20]),
                plsc.load_gather(nwb, [i21]) + plsc.load_gather(ewb, [r21]))
            wb2[pl.ds(0, L)] = v0
            wb2[pl.ds(L, L)] = v1
            pltpu.make_async_copy(
                emb_hbm.at[NBE1.at[i].at[pl.ds(0, KNB)]],
                nb1s[b], s1s[b]).wait()
            wavg = _accumulate(wb, nb1s[b], zacc)
            cur = tuple(A * E1[i, pl.ds(c * L, L)] - (A - 1.0) * wavg[c]
                        for c in range(DC))
            dsq = zero
            for c in range(DC):
                df = cur[c] - E2[i, pl.ds(c * L, L)]
                dsq = dsq + df * df
            d1sq = jnp.sum(dsq)

            pltpu.make_async_copy(
                emb_hbm.at[NBE2.at[i].at[pl.ds(0, KNB)]],
                nb2s[b], s2s[b]).wait()
            wavg2 = _accumulate(wb2, nb2s[b], zacc)
            dsq = zero
            for c in range(DC):
                df = (A * cur[c] - (A - 1.0) * wavg2[c]
                      - E3[i, pl.ds(c * L, L)])
                dsq = dsq + df * df
            d2sq = jnp.sum(dsq)

            iv = jnp.full((L,), i, jnp.int32)
            plsc.store_scatter(d1b, [iv], jnp.full((L,), d1sq), mask=mask0)
            plsc.store_scatter(d2b, [iv], jnp.full((L,), d2sq), mask=mask0)
        return 0

    lax.fori_loop(0, QPW // 2, _pair, 0)

    pltpu.sync_copy(d1b, d1_hbm.at[pl.ds(base, QPW)])
    pltpu.sync_copy(d2b, d2_hbm.at[pl.ds(base, QPW)])


def _tc_finish(d1_ref, d2_ref, o_ref):
    o_ref[0, 0] = (jnp.sum(jnp.sqrt(d1_ref[...])) +
                   jnp.sum(jnp.sqrt(d2_ref[...]))) / BS


def kernel(node_embedding, node_weight, rel_weight, node_neighbors,
           rel_neighbors, e1s, r1s, e2s, r2s, e3s):
    del r2s
    f32 = jnp.float32
    i32 = jnp.int32
    mesh = plsc.VectorSubcoreMesh(core_axis_name="c", subcore_axis_name="s")

    nbr = jnp.concatenate(
        [node_neighbors.astype(i32), rel_neighbors.astype(i32),
         jnp.zeros((N, D - 2 * KNB), i32)], axis=1)

    sc = functools.partial(
        pl.kernel,
        out_type=(jax.ShapeDtypeStruct((BS,), f32),
                  jax.ShapeDtypeStruct((BS,), f32)),
        mesh=mesh,
        compiler_params=pltpu.CompilerParams(needs_layout_passes=False),
        scratch_types=[
            pltpu.VMEM((QPW,), i32),
            pltpu.VMEM((QPW,), i32),
            pltpu.VMEM((QPW,), i32),
            pltpu.VMEM((N + 1,), f32),
            pltpu.VMEM((RP,), f32),
            pltpu.VMEM((BS,), i32),
            pltpu.VMEM((RP,), f32),
            pltpu.VMEM((RP,), f32),
            pltpu.VMEM((QPW, D), f32),
            pltpu.VMEM((QPW, D), f32),
            pltpu.VMEM((QPW, D), f32),
            pltpu.VMEM((QPW, D), i32),
            pltpu.VMEM((QPW, D), i32),
            pltpu.VMEM((KNB, D), f32),
            pltpu.VMEM((KNB, D), f32),
            pltpu.VMEM((KNB, D), f32),
            pltpu.VMEM((KNB, D), f32),
            pltpu.VMEM((KNB,), f32),
            pltpu.VMEM((KNB,), f32),
            pltpu.VMEM((QPW,), f32),
            pltpu.VMEM((QPW,), f32),
            pltpu.SemaphoreType.DMA,
            pltpu.SemaphoreType.DMA,
            pltpu.SemaphoreType.DMA,
            pltpu.SemaphoreType.DMA,
            pltpu.SemaphoreType.DMA,
        ],
    )(_sc_body)

    d1sq, d2sq = sc(node_embedding, node_weight, rel_weight, nbr,
                    e1s.astype(i32), r1s.astype(i32), e2s.astype(i32),
                    e3s.astype(i32))

    out = pl.pallas_call(
        _tc_finish,
        out_shape=jax.ShapeDtypeStruct((1, 1), f32),
        out_specs=pl.BlockSpec(memory_space=pltpu.SMEM),
    )(d1sq.reshape(NW, QPW), d2sq.reshape(NW, QPW))
    return out.reshape(())


# speedup vs baseline: 1.0592x; 1.0592x over previous
import functools

import jax
import jax.numpy as jnp
from jax import lax
from jax.experimental import pallas as pl
from jax.experimental.pallas import tpu as pltpu
from jax.experimental.pallas import tpu_sc as plsc

N = 10000
D = 128
KNB = 32
R = 500
BS = 4096
ST = 4
A = float((1.0 + 1.0 / (KNB * ST)) ** ST)

L = 16
NC = 2
NS = 16
NW = NC * NS
QPW = BS // NW
DC = D // L
RP = 512


def _softmax2(v0, v1):
    m = jnp.maximum(jnp.max(v0), jnp.max(v1))
    x0 = jnp.exp(v0 - m)
    x1 = jnp.exp(v1 - m)
    s = jnp.sum(x0) + jnp.sum(x1)
    return x0 / s, x1 / s


def _sc_body(emb_hbm, nw_hbm, relw_hbm, nbr_hbm,
             e1s_hbm, r1s_hbm, e2s_hbm, e3s_hbm,
             d1_hbm, d2_hbm,
             e1i, e2i, e3i, nwb, relwb, r1b, histb, ewb,
             E1, E2, E3, NBE1, NBE2, nb1a, nb1b, nb2a, nb2b, wb, wb2, d1b, d2b,
             sem0, s1a, s1b, s2a, s2b):
    wid = lax.axis_index("s") * NC + lax.axis_index("c")
    base = wid * QPW

    pltpu.sync_copy(e1s_hbm.at[pl.ds(base, QPW)], e1i)
    pltpu.sync_copy(e2s_hbm.at[pl.ds(base, QPW)], e2i)
    pltpu.sync_copy(e3s_hbm.at[pl.ds(base, QPW)], e3i)
    pltpu.sync_copy(nw_hbm, nwb)
    pltpu.sync_copy(relw_hbm, relwb.at[pl.ds(0, R + 1)])
    pltpu.sync_copy(r1s_hbm, r1b)

    hs = [
        pltpu.async_copy(emb_hbm.at[e1i], E1, sem0),
        pltpu.async_copy(emb_hbm.at[e2i], E2, sem0),
        pltpu.async_copy(emb_hbm.at[e3i], E3, sem0),
        pltpu.async_copy(nbr_hbm.at[e1i], NBE1, sem0),
        pltpu.async_copy(nbr_hbm.at[e2i], NBE2, sem0),
    ]

    zero = jnp.zeros((L,), jnp.float32)
    one = jnp.ones((L,), jnp.float32)
    for j in range(RP // L):
        histb[pl.ds(j * L, L)] = zero

    def _hist_step(j, _):
        idx = r1b[pl.ds(j * L, L)]
        plsc.addupdate_scatter(histb, [idx], one)
        return 0

    lax.fori_loop(0, BS // L, _hist_step, 0)

    for j in range(RP // L):
        sl = pl.ds(j * L, L)
        ewb[sl] = relwb[sl] * (1.0 + histb[sl])

    for h in hs:
        h.wait()

    lane = lax.iota(jnp.int32, L)
    mask0 = lane == 0
    nb1s, nb2s = (nb1a, nb1b), (nb2a, nb2b)
    s1s, s2s = (s1a, s1b), (s2a, s2b)
    UN = 4

    def _fire(q, slot):
        pltpu.make_async_copy(emb_hbm.at[NBE1.at[q].at[pl.ds(0, KNB)]],
                              nb1s[slot], s1s[slot]).start()
        pltpu.make_async_copy(emb_hbm.at[NBE2.at[q].at[pl.ds(0, KNB)]],
                              nb2s[slot], s2s[slot]).start()

    def _accumulate(wref, nbbuf, acc0):
        def _step(j, acc):
            for u in range(UN):
                k = j * UN + u
                wk = plsc.load_gather(wref, [jnp.full((L,), k, jnp.int32)])
                acc = tuple(acc[c] + wk * nbbuf[k, pl.ds(c * L, L)]
                            for c in range(DC))
            return acc
        return lax.fori_loop(0, KNB // UN, _step, acc0)

    zacc = tuple(jnp.zeros((L,), jnp.float32) for _ in range(DC))

    _fire(jnp.int32(0), 0)

    def _pair(g, _):
        for b in (0, 1):
            i = 2 * g + b

            if b == 0:
                _fire(i + 1, 1)
            else:
                @pl.when(g < QPW // 2 - 1)
                def _():
                    _fire(i + 1, 0)

            i10 = NBE1[i, pl.ds(0, L)]
            i11 = NBE1[i, pl.ds(L, L)]
            w0, w1 = _softmax2(plsc.load_gather(nwb, [i10]),
                               plsc.load_gather(nwb, [i11]))
            wb[pl.ds(0, L)] = w0
            wb[pl.ds(L, L)] = w1
            i20 = NBE2[i, pl.ds(0, L)]
            i21 = NBE2[i, pl.ds(L, L)]
            r20 = NBE2[i, pl.ds(KNB, L)]
            r21 = NBE2[i, pl.ds(KNB + L, L)]
            v0, v1 = _softmax2(
                plsc.load_gather(nwb, [i20]) + plsc.load_gather(ewb, [r20]),
                plsc.load_gather(nwb, [i21]) + plsc.load_gather(ewb, [r21]))
            wb2[pl.ds(0, L)] = v0
            wb2[pl.ds(L, L)] = v1
            pltpu.make_async_copy(
                emb_hbm.at[NBE1.at[i].at[pl.ds(0, KNB)]],
                nb1s[b], s1s[b]).wait()
            wavg = _accumulate(wb, nb1s[b], zacc)
            cur = tuple(A * E1[i, pl.ds(c * L, L)] - (A - 1.0) * wavg[c]
                        for c in range(DC))
            dsq = zero
            for c in range(DC):
                df = cur[c] - E2[i, pl.ds(c * L, L)]
                dsq = dsq + df * df
            d1sq = jnp.sum(dsq)

            pltpu.make_async_copy(
                emb_hbm.at[NBE2.at[i].at[pl.ds(0, KNB)]],
                nb2s[b], s2s[b]).wait()
            wavg2 = _accumulate(wb2, nb2s[b], zacc)
            dsq = zero
            for c in range(DC):
                df = (A * cur[c] - (A - 1.0) * wavg2[c]
                      - E3[i, pl.ds(c * L, L)])
                dsq = dsq + df * df
            d2sq = jnp.sum(dsq)

            iv = jnp.full((L,), i, jnp.int32)
            plsc.store_scatter(d1b, [iv], jnp.full((L,), d1sq), mask=mask0)
            plsc.store_scatter(d2b, [iv], jnp.full((L,), d2sq), mask=mask0)
        return 0

    lax.fori_loop(0, QPW // 2, _pair, 0)

    pltpu.sync_copy(d1b, d1_hbm.at[pl.ds(base, QPW)])
    pltpu.sync_copy(d2b, d2_hbm.at[pl.ds(base, QPW)])


def _tc_finish(d1_ref, d2_ref, o_ref):
    o_ref[0, 0] = (jnp.sum(jnp.sqrt(d1_ref[...])) +
                   jnp.sum(jnp.sqrt(d2_ref[...]))) / BS


def kernel(node_embedding, node_weight, rel_weight, node_neighbors,
           rel_neighbors, e1s, r1s, e2s, r2s, e3s):
    del r2s
    f32 = jnp.float32
    i32 = jnp.int32
    mesh = plsc.VectorSubcoreMesh(core_axis_name="c", subcore_axis_name="s")

    nbr = jnp.concatenate(
        [node_neighbors.astype(i32), rel_neighbors.astype(i32),
         jnp.zeros((N, D - 2 * KNB), i32)], axis=1)

    sc = functools.partial(
        pl.kernel,
        out_type=(jax.ShapeDtypeStruct((BS,), f32),
                  jax.ShapeDtypeStruct((BS,), f32)),
        mesh=mesh,
        compiler_params=pltpu.CompilerParams(needs_layout_passes=False),
        scratch_types=[
            pltpu.VMEM((QPW,), i32),
            pltpu.VMEM((QPW,), i32),
            pltpu.VMEM((QPW,), i32),
            pltpu.VMEM((N + 1,), f32),
            pltpu.VMEM((RP,), f32),
            pltpu.VMEM((BS,), i32),
            pltpu.VMEM((RP,), f32),
            pltpu.VMEM((RP,), f32),
            pltpu.VMEM((QPW, D), f32),
            pltpu.VMEM((QPW, D), f32),
            pltpu.VMEM((QPW, D), f32),
            pltpu.VMEM((QPW, D), i32),
            pltpu.VMEM((QPW, D), i32),
            pltpu.VMEM((KNB, D), f32),
            pltpu.VMEM((KNB, D), f32),
            pltpu.VMEM((KNB, D), f32),
            pltpu.VMEM((KNB, D), f32),
            pltpu.VMEM((KNB,), f32),
            pltpu.VMEM((KNB,), f32),
            pltpu.VMEM((QPW,), f32),
            pltpu.VMEM((QPW,), f32),
            pltpu.SemaphoreType.DMA,
            pltpu.SemaphoreType.DMA,
            pltpu.SemaphoreType.DMA,
            pltpu.SemaphoreType.DMA,
            pltpu.SemaphoreType.DMA,
        ],
    )(_sc_body)

    d1sq, d2sq = sc(node_embedding, node_weight, rel_weight, nbr,
                    e1s.astype(i32), r1s.astype(i32), e2s.astype(i32),
                    e3s.astype(i32))

    out = pl.pallas_call(
        _tc_finish,
        out_shape=jax.ShapeDtypeStruct((1, 1), f32),
        out_specs=pl.BlockSpec(memory_space=pltpu.SMEM),
    )(d1sq.reshape(NW, QPW), d2sq.reshape(NW, QPW))
    return out.reshape(())
